# baseline (speedup 1.0000x reference)
import functools

import jax
import jax.numpy as jnp
from jax import lax
from jax.experimental import pallas as pl
from jax.experimental.pallas import tpu as pltpu
from jax.experimental.pallas import tpu_sc as plsc

_mesh = plsc.VectorSubcoreMesh(core_axis_name="c", subcore_axis_name="s")

_ROW_MAP = ((0, 0), (6, 1), (1, 2))


@functools.partial(
    pl.kernel,
    mesh=_mesh,
    out_type=jax.ShapeDtypeStruct((2, 3, 128), jnp.float32),
    scratch_types=[pltpu.VMEM((3, 128), jnp.float32)],
)
def _gather_rows(x_hbm, out_hbm, buf):
    wid = lax.axis_index("s") * 2 + lax.axis_index("c")

    @pl.when(wid == 0)
    def _():
        pltpu.sync_copy(x_hbm.at[0, pl.ds(2, 3)], buf)
        pltpu.sync_copy(buf, out_hbm.at[0])

    for w, (src_j, dst_b) in enumerate(_ROW_MAP, start=1):

        @pl.when(wid == w)
        def _(src_j=src_j, dst_b=dst_b):
            row = buf.at[pl.ds(0, 1)]
            pltpu.sync_copy(x_hbm.at[1, pl.ds(src_j, 1)], row)
            pltpu.sync_copy(row, out_hbm.at[1, pl.ds(dst_b, 1)])


def kernel(x):
    return _gather_rows(x)
